# baseline (speedup 1.0000x reference)
import math

import jax
import jax.numpy as jnp
from jax import lax
from jax.experimental import pallas as pl
from jax.experimental.pallas import tpu as pltpu
from jax.experimental.pallas import tpu_sc as plsc

D = 128
PAD = 5000
SCALE = math.sqrt(float(D))
NC, NS, LANES = 2, 16, 16
NW = NC * NS
GS = ((0, 128),)
C = 128


def _body(x_hbm, m_hbm, i_hbm, pe_hbm, out_hbm,
          idx0, idx1, idx2, idx3, msk0, msk1, msk2, msk3,
          xv0, xv1, xv2, xv3, pe_sh,
          sin0, sin1, sin2, sin3, sg0, sg1, sg2, sg3,
          so0, so1, so2, so3):
    idx_v = (idx0, idx1, idx2, idx3)
    msk_v = (msk0, msk1, msk2, msk3)
    x_v = (xv0, xv1, xv2, xv3)
    sin = (sin0, sin1, sin2, sin3)
    sg = (sg0, sg1, sg2, sg3)
    so = (so0, so1, so2, so3)
    n = x_hbm.shape[0]
    rows_per_w = n // NW
    steps = rows_per_w // C
    sid = lax.axis_index("s")
    wid = sid * NC + lax.axis_index("c")
    base0 = wid * rows_per_w

    @pl.when(sid == 0)
    def _stage():
        pltpu.sync_copy(pe_hbm, pe_sh)

    plsc.subcore_barrier()

    def issue_in(p, t):
        base = base0 + t * C
        pltpu.async_copy(i_hbm.at[pl.ds(base, C)], idx_v[p], sin[p])
        pltpu.async_copy(m_hbm.at[pl.ds(base, C)], msk_v[p], sin[p])
        pltpu.async_copy(x_hbm.at[pl.ds(base, C), :], x_v[p], sin[p])

    def drain_in(p):
        pltpu.make_async_copy(i_hbm.at[pl.ds(0, C)], idx_v[p], sin[p]).wait()
        pltpu.make_async_copy(m_hbm.at[pl.ds(0, C)], msk_v[p], sin[p]).wait()
        pltpu.make_async_copy(x_hbm.at[pl.ds(0, C), :], x_v[p], sin[p]).wait()

    def prep(p):
        for j in range(C // LANES):
            sl = pl.ds(j * LANES, LANES)
            iv = idx_v[p][sl]
            mv = msk_v[p][sl]
            idx_v[p][sl] = jnp.where(mv != 0, PAD, jnp.minimum(iv, PAD))
        @plsc.parallel_loop(0, C, 1, unroll=4)
        def _row(i):
            for j in range(D // LANES):
                sl = pl.ds(j * LANES, LANES)
                x_v[p][i, sl] = SCALE * x_v[p][i, sl]

    def issue_gather(p):
        for o, g in GS:
            pltpu.async_copy(pe_sh.at[idx_v[p].at[pl.ds(o, g)]],
                             x_v[p].at[pl.ds(o, g)], sg[p], add=True)

    def drain_gather(p):
        for o, g in GS:
            pltpu.make_async_copy(pe_sh.at[idx_v[p].at[pl.ds(o, g)]],
                                  x_v[p].at[pl.ds(o, g)], sg[p]).wait()

    def issue_out(p, t):
        base = base0 + t * C
        pltpu.async_copy(x_v[p], out_hbm.at[pl.ds(base, C), :], so[p])

    def drain_out(p):
        pltpu.make_async_copy(x_v[p], out_hbm.at[pl.ds(0, C), :], so[p]).wait()

    issue_in(0, 0)
    drain_in(0)
    prep(0)
    issue_gather(0)
    issue_in(1, 1)

    def iteration(t, p):
        q = (p + 1) % 4
        f = (p + 2) % 4

        @pl.when(t >= 2)
        def _free():
            drain_out(f)

        issue_in(f, t + 2)
        drain_in(q)
        prep(q)
        issue_gather(q)
        drain_gather(p)
        issue_out(p, t)

    nloop = (steps - 4) // 4
    @pl.loop(0, nloop)
    def _quad(u):
        for e in range(4):
            iteration(4 * u + e, e)

    for t in range(4 * nloop, steps):
        p = t % 4
        q = (p + 1) % 4
        f = (p + 2) % 4
        if t >= 2:
            drain_out(f)
        if t + 2 < steps:
            issue_in(f, t + 2)
        if t + 1 < steps:
            drain_in(q)
            prep(q)
            issue_gather(q)
        drain_gather(p)
        issue_out(p, t)
    drain_out((steps - 2) % 4)
    drain_out((steps - 1) % 4)


def kernel(x, mask, indices, pe):
    b, l, d = x.shape
    n = b * l
    x2 = x.reshape(n, d)
    m2 = mask.reshape(n).astype(jnp.int32)
    i2 = indices.reshape(n).astype(jnp.int32)

    mesh = plsc.VectorSubcoreMesh(core_axis_name="c", subcore_axis_name="s")
    out = pl.kernel(
        _body,
        out_type=jax.ShapeDtypeStruct((n, d), jnp.float32),
        mesh=mesh,
        scratch_types=[
        ] + [pltpu.VMEM((C,), jnp.int32)] * 8
          + [pltpu.VMEM((C, D), jnp.float32)] * 4
          + [pltpu.VMEM_SHARED((PAD + 1, D), jnp.float32)]
          + [pltpu.SemaphoreType.DMA] * 12,
    )(x2, m2, i2, pe)
    return out.reshape(b, l, d)


# speedup vs baseline: 1.2270x; 1.2270x over previous
import math

import jax
import jax.numpy as jnp
from jax import lax
from jax.experimental import pallas as pl
from jax.experimental.pallas import tpu as pltpu
from jax.experimental.pallas import tpu_sc as plsc

D = 128
PAD = 5000
SCALE = math.sqrt(float(D))
NC, NS, LANES = 2, 16, 16
NW = NC * NS
GS = ((0, 80), (80, 80))
C = 160


def _body(x_hbm, m_hbm, i_hbm, pe_hbm, out_hbm,
          idx0, idx1, idx2, msk0, msk1, msk2, xv0, xv1, xv2, pe_sh,
          sin0, sin1, sin2, sg0, sg1, sg2, so0, so1, so2):
    idx_v = (idx0, idx1, idx2)
    msk_v = (msk0, msk1, msk2)
    x_v = (xv0, xv1, xv2)
    sin = (sin0, sin1, sin2)
    sg = (sg0, sg1, sg2)
    so = (so0, so1, so2)
    n = x_hbm.shape[0]
    rows_per_w = n // NW
    steps = rows_per_w // C
    sid = lax.axis_index("s")
    wid = sid * NC + lax.axis_index("c")
    base0 = wid * rows_per_w

    @pl.when(sid == 0)
    def _stage():
        pltpu.sync_copy(pe_hbm, pe_sh)

    plsc.subcore_barrier()

    def issue_in(p, t):
        base = base0 + t * C
        pltpu.async_copy(i_hbm.at[pl.ds(base, C)], idx_v[p], sin[p])
        pltpu.async_copy(m_hbm.at[pl.ds(base, C)], msk_v[p], sin[p])
        pltpu.async_copy(x_hbm.at[pl.ds(base, C), :], x_v[p], sin[p])

    def drain_in(p):
        pltpu.make_async_copy(i_hbm.at[pl.ds(0, C)], idx_v[p], sin[p]).wait()
        pltpu.make_async_copy(m_hbm.at[pl.ds(0, C)], msk_v[p], sin[p]).wait()
        pltpu.make_async_copy(x_hbm.at[pl.ds(0, C), :], x_v[p], sin[p]).wait()

    def prep(p):
        for j in range(C // LANES):
            sl = pl.ds(j * LANES, LANES)
            iv = idx_v[p][sl]
            mv = msk_v[p][sl]
            idx_v[p][sl] = jnp.where(mv != 0, PAD, jnp.minimum(iv, PAD))
        @plsc.parallel_loop(0, C, 1, unroll=4)
        def _row(i):
            for j in range(D // LANES):
                sl = pl.ds(j * LANES, LANES)
                x_v[p][i, sl] = SCALE * x_v[p][i, sl]

    def issue_gather(p):
        for o, g in GS:
            pltpu.async_copy(pe_sh.at[idx_v[p].at[pl.ds(o, g)]],
                             x_v[p].at[pl.ds(o, g)], sg[p], add=True)

    def drain_gather(p):
        for o, g in GS:
            pltpu.make_async_copy(pe_sh.at[idx_v[p].at[pl.ds(o, g)]],
                                  x_v[p].at[pl.ds(o, g)], sg[p]).wait()

    def issue_out(p, t):
        base = base0 + t * C
        pltpu.async_copy(x_v[p], out_hbm.at[pl.ds(base, C), :], so[p])

    def drain_out(p):
        pltpu.make_async_copy(x_v[p], out_hbm.at[pl.ds(0, C), :], so[p]).wait()

    issue_in(0, 0)
    drain_in(0)
    prep(0)
    issue_gather(0)
    issue_in(1, 1)

    def iteration(t, p):
        q = (p + 1) % 3
        r = (p + 2) % 3

        @pl.when(t >= 1)
        def _free():
            drain_out(r)

        @pl.when(t + 2 < steps)
        def _prefetch():
            issue_in(r, t + 2)

        drain_in(q)
        prep(q)
        issue_gather(q)
        drain_gather(p)
        issue_out(p, t)

    @pl.loop(0, steps // 3)
    def _triple(u):
        for e in range(3):
            iteration(3 * u + e, e)

    for t in range(3 * (steps // 3), steps):
        p = t % 3
        q = (p + 1) % 3
        drain_out((t + 2) % 3)
        if t + 1 < steps:
            drain_in(q)
            prep(q)
            issue_gather(q)
        drain_gather(p)
        issue_out(p, t)
    drain_out((steps - 1) % 3)


def kernel(x, mask, indices, pe):
    b, l, d = x.shape
    n = b * l
    x2 = x.reshape(n, d)
    m2 = mask.reshape(n).astype(jnp.int32)
    i2 = indices.reshape(n).astype(jnp.int32)

    mesh = plsc.VectorSubcoreMesh(core_axis_name="c", subcore_axis_name="s")
    out = pl.kernel(
        _body,
        out_type=jax.ShapeDtypeStruct((n, d), jnp.float32),
        mesh=mesh,
        scratch_types=(
            [pltpu.VMEM((C,), jnp.int32)] * 6
            + [pltpu.VMEM((C, D), jnp.float32)] * 3
            + [pltpu.VMEM_SHARED((PAD + 1, D), jnp.float32)]
            + [pltpu.SemaphoreType.DMA] * 9),
    )(x2, m2, i2, pe)
    return out.reshape(b, l, d)
